# Initial kernel scaffold; baseline (speedup 1.0000x reference)
import jax
import jax.numpy as jnp
from jax.experimental import pallas as pl


def kernel(x, pos_table):
    raise NotImplementedError("write your pallas kernel here")


# speedup vs baseline: 1.6756x; 1.6756x over previous
import jax
import jax.numpy as jnp
from jax.experimental import pallas as pl


_SBLK = 512


def _add_body(x_ref, pos_ref, o_ref):
    o_ref[...] = x_ref[...] + pos_ref[...]


def kernel(x, pos_table):
    B, S, D = x.shape
    return pl.pallas_call(
        _add_body,
        grid=(S // _SBLK, B),
        in_specs=[
            pl.BlockSpec((1, _SBLK, D), lambda i, b: (b, i, 0)),
            pl.BlockSpec((_SBLK, D), lambda i, b: (i, 0)),
        ],
        out_specs=pl.BlockSpec((1, _SBLK, D), lambda i, b: (b, i, 0)),
        out_shape=jax.ShapeDtypeStruct(x.shape, x.dtype),
    )(x, pos_table[:S])
